# baseline (speedup 1.0000x reference)
import functools

import jax
import jax.numpy as jnp
from jax import lax
from jax.experimental import pallas as pl
from jax.experimental.pallas import tpu as pltpu
from jax.experimental.pallas import tpu_sc as plsc

B = 1024
MAXLEN = 1024
EMB = 128
VOCAB = 21

NC = 2
NS = 16
NW = NC * NS
NTOK = B * MAXLEN
TOK_PER_W = NTOK // NW
CH = 128
NCH = TOK_PER_W // CH
NBUF = 4

@functools.cache
def _make_emb_sc():
    mesh = plsc.VectorSubcoreMesh(
        core_axis_name="c", subcore_axis_name="s", num_cores=NC, num_subcores=NS
    )
    return functools.partial(
        pl.kernel,
        out_type=jax.ShapeDtypeStruct((NTOK, EMB), jnp.float32),
        mesh=mesh,
        scratch_types=[
            pltpu.VMEM((NCH, CH), jnp.int32),
            pltpu.VMEM((NBUF, CH, EMB), jnp.float32),
            pltpu.VMEM_SHARED((VOCAB, EMB), jnp.float32),
            [pltpu.SemaphoreType.DMA] * NBUF,
            [pltpu.SemaphoreType.DMA] * NBUF,
        ],
    )(_emb_sc_body)


def _emb_sc_body(tok_hbm, table_hbm, out_hbm, tok_v, rows_v, table_v, gs, os):
    wid = lax.axis_index("s") * NC + lax.axis_index("c")
    chunk0 = wid * NCH

    @pl.when(lax.axis_index("s") == 0)
    def _():
        pltpu.sync_copy(table_hbm, table_v)

    pltpu.sync_copy(tok_hbm.at[pl.ds(chunk0, NCH)], tok_v)
    plsc.subcore_barrier()

    def start_gather(c, b):
        pltpu.async_copy(table_v.at[tok_v.at[c]], rows_v.at[b], gs[b])

    def wait_gather(c, b):
        pltpu.make_async_copy(table_v.at[tok_v.at[c]], rows_v.at[b], gs[b]).wait()

    def start_store(c, b):
        pltpu.async_copy(
            rows_v.at[b], out_hbm.at[pl.ds((chunk0 + c) * CH, CH)], os[b]
        )

    def wait_store(b):
        pltpu.make_async_copy(
            rows_v.at[b], out_hbm.at[pl.ds(chunk0 * CH, CH)], os[b]
        ).wait()

    for b in range(NBUF):
        start_gather(b, b)

    @pl.loop(0, NCH // NBUF - 1)
    def _(i):
        c0 = i * NBUF
        for b in range(NBUF):
            wait_gather(c0 + b, b)
            start_store(c0 + b, b)
        for b in range(NBUF):
            wait_store(b)
            start_gather(c0 + NBUF + b, b)

    c0 = NCH - NBUF
    for b in range(NBUF):
        wait_gather(c0 + b, b)
        start_store(c0 + b, b)
    for b in range(NBUF):
        wait_store(b)


def _count_body(tok_ref, out_ref):
    t = tok_ref[...].reshape(8, 128, MAXLEN)
    out_ref[...] = jnp.sum((t != 0).astype(jnp.int32), axis=2)


_count_tc = pl.pallas_call(
    _count_body,
    out_shape=jax.ShapeDtypeStruct((8, 128), jnp.int32),
)


def kernel(tokens, table):
    tok2d = tokens.reshape(NTOK // CH, CH)
    emb_flat = _make_emb_sc()(tok2d, table)
    emb = emb_flat.reshape(B, MAXLEN, EMB)
    pro_lens = _count_tc(tokens).reshape(B)
    return emb, pro_lens
